# speedup vs baseline: 3.0874x; 3.0874x over previous
import jax
import jax.numpy as jnp
from jax.experimental import pallas as pl
from jax.experimental.pallas import tpu as pltpu
from jax.experimental.pallas import tpu_sc as plsc

_WINDOW = 128


def kernel(indices, table):
    batch, seq = indices.shape
    num_rows, dim = table.shape
    num_indices = batch * seq
    flat_idx = indices.reshape(1, num_indices).astype(jnp.int32)

    mesh = plsc.VectorSubcoreMesh(core_axis_name="core",
                                  subcore_axis_name="subcore")

    @pl.kernel(
        out_type=jax.ShapeDtypeStruct((num_indices, dim), table.dtype),
        mesh=mesh,
    )
    def gather_kernel(table_hbm, idx_hbm, out_hbm):
        def body(idx_vmem, out_vmem):
            pltpu.sync_copy(table_hbm.at[idx_vmem.at[0]], out_vmem)

        pltpu.emit_pipeline(
            body,
            grid=(num_indices // _WINDOW,),
            in_specs=[pl.BlockSpec((1, _WINDOW), index_map=lambda i: (0, i))],
            out_specs=[pl.BlockSpec((_WINDOW, dim), index_map=lambda i: (i, 0))],
            core_axis_name=("core", "subcore"),
            dimension_semantics=(pltpu.PARALLEL,),
        )(idx_hbm, out_hbm)

    out = gather_kernel(table, flat_idx)
    return out.reshape(batch, seq, dim)


# speedup vs baseline: 3.2943x; 1.0670x over previous
import jax
import jax.numpy as jnp
from jax.experimental import pallas as pl
from jax.experimental.pallas import tpu as pltpu
from jax.experimental.pallas import tpu_sc as plsc

_WINDOW = 256


def kernel(indices, table):
    batch, seq = indices.shape
    num_rows, dim = table.shape
    num_indices = batch * seq
    flat_idx = indices.reshape(1, num_indices).astype(jnp.int32)

    mesh = plsc.VectorSubcoreMesh(core_axis_name="core",
                                  subcore_axis_name="subcore")

    @pl.kernel(
        out_type=jax.ShapeDtypeStruct((num_indices, dim), table.dtype),
        mesh=mesh,
    )
    def gather_kernel(table_hbm, idx_hbm, out_hbm):
        def body(idx_vmem, out_vmem):
            pltpu.sync_copy(table_hbm.at[idx_vmem.at[0]], out_vmem)

        pltpu.emit_pipeline(
            body,
            grid=(num_indices // _WINDOW,),
            in_specs=[pl.BlockSpec((1, _WINDOW), index_map=lambda i: (0, i))],
            out_specs=[pl.BlockSpec((_WINDOW, dim), index_map=lambda i: (i, 0))],
            core_axis_name=("core", "subcore"),
            dimension_semantics=(pltpu.PARALLEL,),
        )(idx_hbm, out_hbm)

    out = gather_kernel(table, flat_idx)
    return out.reshape(batch, seq, dim)


# speedup vs baseline: 3.3264x; 1.0097x over previous
import jax
import jax.numpy as jnp
from jax import lax
from jax.experimental import pallas as pl
from jax.experimental.pallas import tpu as pltpu
from jax.experimental.pallas import tpu_sc as plsc

_NCORES = 2
_NSUB = 16
_NWORKERS = _NCORES * _NSUB
_CHUNK = 128
_NSLOTS = 6
_LOOKAHEAD = 4


def kernel(indices, table):
    batch, seq = indices.shape
    num_rows, dim = table.shape
    num_indices = batch * seq
    per_worker = num_indices // _NWORKERS
    nchunks = per_worker // _CHUNK
    flat_idx = indices.reshape(num_indices).astype(jnp.int32)

    mesh = plsc.VectorSubcoreMesh(core_axis_name="c", subcore_axis_name="s")

    sem_types = [pltpu.SemaphoreType.DMA] * (2 * _NSLOTS)

    @pl.kernel(
        out_type=jax.ShapeDtypeStruct((num_indices, dim), table.dtype),
        mesh=mesh,
        scratch_types=[
            pltpu.VMEM((per_worker,), jnp.int32),
            pltpu.VMEM((_NSLOTS, _CHUNK, dim), table.dtype),
        ] + sem_types,
    )
    def gather_kernel(table_hbm, idx_hbm, out_hbm, idx_v, rows_v, *sems):
        g_sems = sems[:_NSLOTS]
        s_sems = sems[_NSLOTS:]
        wid = lax.axis_index("s") * _NCORES + lax.axis_index("c")
        base = wid * per_worker

        pltpu.sync_copy(idx_hbm.at[pl.ds(base, per_worker)], idx_v)

        def fire(c, slot):
            pltpu.async_copy(
                table_hbm.at[idx_v.at[pl.ds(c * _CHUNK, _CHUNK)]],
                rows_v.at[slot], g_sems[slot])

        def wait_gather(slot):
            pltpu.make_async_copy(
                table_hbm.at[idx_v.at[pl.ds(0, _CHUNK)]],
                rows_v.at[slot], g_sems[slot]).wait()

        def store(c, slot):
            pltpu.async_copy(
                rows_v.at[slot],
                out_hbm.at[pl.ds(base + c * _CHUNK, _CHUNK)], s_sems[slot])

        def wait_store(slot):
            pltpu.make_async_copy(
                rows_v.at[slot],
                out_hbm.at[pl.ds(base, _CHUNK)], s_sems[slot]).wait()

        for c in range(_LOOKAHEAD):
            fire(c, c % _NSLOTS)

        def chunk_body(c, slot, fire_next, wait_prev_store):
            wait_gather(slot)
            store(c, slot)
            if fire_next:
                nxt_slot = (slot + _LOOKAHEAD) % _NSLOTS
                if wait_prev_store:
                    wait_store(nxt_slot)
                fire(c + _LOOKAHEAD, nxt_slot)

        chunk_body(0, 0, True, False)
        chunk_body(1, 1, True, False)

        steady = ((nchunks - _LOOKAHEAD - 2) // _NSLOTS) * _NSLOTS

        @pl.loop(2, 2 + steady, step=_NSLOTS)
        def _(c0):
            for j in range(_NSLOTS):
                chunk_body(c0 + j, (2 + j) % _NSLOTS, True, True)

        c = 2 + steady
        while c + _LOOKAHEAD < nchunks:
            chunk_body(c, c % _NSLOTS, True, True)
            c += 1
        while c < nchunks:
            chunk_body(c, c % _NSLOTS, False, False)
            c += 1

        for s in range(_NSLOTS):
            wait_store(s)

    out = gather_kernel(table, flat_idx)
    return out.reshape(batch, seq, dim)


# speedup vs baseline: 5.9341x; 1.7839x over previous
import jax
import jax.numpy as jnp
from jax import lax
from jax.experimental import pallas as pl
from jax.experimental.pallas import tpu as pltpu
from jax.experimental.pallas import tpu_sc as plsc

_NCORES = 2
_NSUB = 16
_NWORKERS = _NCORES * _NSUB
_NSLOTS = 6
_LOOKAHEAD = 4


def kernel(indices, table):
    batch, seq = indices.shape
    num_rows, dim = table.shape
    per_worker = batch // _NWORKERS
    idx32 = indices.astype(jnp.int32)

    mesh = plsc.VectorSubcoreMesh(core_axis_name="c", subcore_axis_name="s")

    sem_types = [pltpu.SemaphoreType.DMA] * (2 * _NSLOTS)

    @pl.kernel(
        out_type=jax.ShapeDtypeStruct((batch, seq, dim), table.dtype),
        mesh=mesh,
        scratch_types=[
            pltpu.VMEM((per_worker, seq), jnp.int32),
            pltpu.VMEM((_NSLOTS, seq, dim), table.dtype),
        ] + sem_types,
    )
    def gather_kernel(table_hbm, idx_hbm, out_hbm, idx_v, rows_v, *sems):
        g_sems = sems[:_NSLOTS]
        s_sems = sems[_NSLOTS:]
        wid = lax.axis_index("s") * _NCORES + lax.axis_index("c")
        base = wid * per_worker

        pltpu.sync_copy(idx_hbm.at[pl.ds(base, per_worker)], idx_v)

        def fire(c, slot):
            pltpu.async_copy(
                table_hbm.at[idx_v.at[c]], rows_v.at[slot], g_sems[slot])

        def wait_gather(slot):
            pltpu.make_async_copy(
                table_hbm.at[idx_v.at[0]],
                rows_v.at[slot], g_sems[slot]).wait()

        def store(c, slot):
            pltpu.async_copy(
                rows_v.at[slot], out_hbm.at[base + c], s_sems[slot])

        def wait_store(slot):
            pltpu.make_async_copy(
                rows_v.at[slot], out_hbm.at[base], s_sems[slot]).wait()

        for c in range(_LOOKAHEAD):
            fire(c, c % _NSLOTS)

        def chunk_body(c, slot, fire_next, wait_prev_store):
            wait_gather(slot)
            store(c, slot)
            if fire_next:
                nxt_slot = (slot + _LOOKAHEAD) % _NSLOTS
                if wait_prev_store:
                    wait_store(nxt_slot)
                fire(c + _LOOKAHEAD, nxt_slot)

        chunk_body(0, 0, True, False)
        chunk_body(1, 1, True, False)

        steady = ((per_worker - _LOOKAHEAD - 2) // _NSLOTS) * _NSLOTS

        @pl.loop(2, 2 + steady, step=_NSLOTS)
        def _(c0):
            for j in range(_NSLOTS):
                chunk_body(c0 + j, (2 + j) % _NSLOTS, True, True)

        c = 2 + steady
        while c + _LOOKAHEAD < per_worker:
            chunk_body(c, c % _NSLOTS, True, True)
            c += 1
        while c < per_worker:
            chunk_body(c, c % _NSLOTS, False, False)
            c += 1

        for s in range(_NSLOTS):
            wait_store(s)

    return gather_kernel(table, idx32)
